# baseline (speedup 1.0000x reference)
import functools

import jax
import jax.numpy as jnp
from jax import lax
from jax.experimental import pallas as pl
from jax.experimental.pallas import tpu as pltpu
from jax.experimental.pallas import tpu_sc as plsc

N = 10000
K = 32
D_s = 128
D_id = 32
HD = 64
FF = 256

NC = 2
NS = 16
NW = NC * NS

N_PAD = 10240
E_PAD = N_PAD * K
EPW = E_PAD // NW
CH = 128
CHUNKS = EPW // CH
IDXROWS = E_PAD // CH

BN1 = 512
BN2 = 256

@functools.cache
def _sc_mesh():
    return plsc.VectorSubcoreMesh(core_axis_name="c", subcore_axis_name="s",
                                  num_cores=NC, num_subcores=NS)


def _rms(x):
    return x * lax.rsqrt(jnp.mean(x * x, axis=-1, keepdims=True) + 1e-6)


def _tc1_body(s_ref, cid_ref, wc1a, wc1b, bc1, wc2, bc2,
              wq1a, wq1b, bq1, wq2, bq2, wk1, bk1, wk2, bk2,
              m_ref, q_ref, k_ref):
    x = s_ref[...]
    cid = cid_ref[...]
    sn = _rms(x)
    h = jax.nn.gelu(sn @ wc1a[...] + cid @ wc1b[...] + bc1[...])
    m_ref[...] = h @ wc2[...] + bc2[...]
    hq = jax.nn.gelu(x @ wq1a[...] + cid @ wq1b[...] + bq1[...])
    q_ref[...] = hq @ wq2[...] + bq2[...]
    hk = jax.nn.gelu(cid @ wk1[...] + bk1[...])
    k_ref[...] = hk @ wk2[...] + bk2[...]


def _tc1(s_pad, cid_pad, wc1a, wc1b, bc1, wc2, bc2,
         wq1a, wq1b, bq1, wq2, bq2, wk1, bk1, wk2, bk2):
    g = N_PAD // BN1
    row = lambda i: (i, 0)
    full = lambda i: (0, 0)
    wspec = lambda a: pl.BlockSpec(a.shape, full)
    return pl.pallas_call(
        _tc1_body,
        grid=(g,),
        in_specs=[pl.BlockSpec((BN1, D_s), row), pl.BlockSpec((BN1, D_id), row)]
        + [wspec(a) for a in (wc1a, wc1b, bc1, wc2, bc2,
                              wq1a, wq1b, bq1, wq2, bq2, wk1, bk1, wk2, bk2)],
        out_specs=[pl.BlockSpec((BN1, D_s), row),
                   pl.BlockSpec((BN1, HD), row),
                   pl.BlockSpec((BN1, HD), row)],
        out_shape=[jax.ShapeDtypeStruct((N_PAD, D_s), jnp.float32),
                   jax.ShapeDtypeStruct((N_PAD, HD), jnp.float32),
                   jax.ShapeDtypeStruct((N_PAD, HD), jnp.float32)],
    )(s_pad, cid_pad, wc1a, wc1b, bc1, wc2, bc2,
      wq1a, wq1b, bq1, wq2, bq2, wk1, bk1, wk2, bk2)


def _sc_gather_body(k_hbm, idx_hbm, out_hbm, idx_v, rows_v, sem):
    wid = lax.axis_index("s") * NC + lax.axis_index("c")
    base_idx_row = wid * CHUNKS
    pltpu.sync_copy(idx_hbm.at[pl.ds(base_idx_row, CHUNKS)], idx_v)

    G = 4

    def step(i, _):
        descs = []
        for g in range(G):
            j = i * G + g
            d = pltpu.async_copy(k_hbm.at[idx_v.at[j]],
                                 rows_v.at[pl.ds(g * CH, CH)], sem)
            descs.append(d)
        for d in descs:
            d.wait()
        pltpu.sync_copy(
            rows_v, out_hbm.at[pl.ds(wid * EPW + i * (G * CH), G * CH)])
        return 0

    lax.fori_loop(0, CHUNKS // G, step, 0)


@functools.cache
def _sc_gather_kernel():
    return pl.kernel(
        _sc_gather_body,
        out_type=jax.ShapeDtypeStruct((E_PAD, HD), jnp.float32),
        mesh=_sc_mesh(),
        scratch_types=[pltpu.VMEM((CHUNKS, CH), jnp.int32),
                       pltpu.VMEM((4 * CH, HD), jnp.float32),
                       pltpu.SemaphoreType.DMA],
        compiler_params=pltpu.CompilerParams(use_tc_tiling_on_sc=False),
    )


def _sc_gather(kvec, idx_flat):
    return _sc_gather_kernel()(kvec, idx_flat)


def _tc3_body(q_ref, knbr_ref, eb_ref, m_ref, msgs_ref):
    i = pl.program_id(0)
    qb = q_ref[...]
    m = m_ref[...]
    kn = knbr_ref[...]
    eb = eb_ref[...]
    row = i * BN2 + lax.broadcasted_iota(jnp.int32, (BN2, 1), 0)
    valid = row < N
    for k in range(K):
        sc = jnp.sum(qb * kn[k], axis=-1, keepdims=True) + eb[:, k:k + 1]
        w = jnp.where(valid, jax.nn.sigmoid(sc), 0.0)
        msgs_ref[k, :, :] = w * m


def _tc3(q, knbr_t, eb_pad, m):
    g = N_PAD // BN2
    row = lambda i: (i, 0)
    return pl.pallas_call(
        _tc3_body,
        grid=(g,),
        in_specs=[pl.BlockSpec((BN2, HD), row),
                  pl.BlockSpec((K, BN2, HD), lambda i: (0, i, 0)),
                  pl.BlockSpec((BN2, K), row),
                  pl.BlockSpec((BN2, D_s), row)],
        out_specs=pl.BlockSpec((K, BN2, D_s), lambda i: (0, i, 0)),
        out_shape=jax.ShapeDtypeStruct((K, N_PAD, D_s), jnp.float32),
    )(q, knbr_t, eb_pad, m)


def _sc_scatter_body(msgs_hbm, dst_hbm, zer_hbm, out_hbm,
                     dst_v, msgs_v, sem, acc):
    c = lax.axis_index("c")
    sid = lax.axis_index("s")
    wid = sid * NC + c
    rows_per_tile = N_PAD // NS
    zbase = sid * rows_per_tile

    pltpu.sync_copy(dst_hbm.at[pl.ds(wid * CHUNKS, CHUNKS)], dst_v)
    for j in range(rows_per_tile // CH):
        pltpu.sync_copy(zer_hbm, acc.at[pl.ds(zbase + j * CH, CH)])
    plsc.subcore_barrier()

    def step(j, _):
        pltpu.sync_copy(msgs_hbm.at[pl.ds(wid * EPW + j * CH, CH)], msgs_v)
        pltpu.sync_copy(msgs_v, acc.at[dst_v.at[j]], add=True)
        return 0

    lax.fori_loop(0, CHUNKS, step, 0)
    plsc.subcore_barrier()

    for j in range(rows_per_tile // CH):
        r = zbase + j * CH
        pltpu.sync_copy(acc.at[pl.ds(r, CH)],
                        out_hbm.at[pl.ds(c * N_PAD + r, CH)])


@functools.cache
def _sc_scatter_kernel():
    return pl.kernel(
        _sc_scatter_body,
        out_type=jax.ShapeDtypeStruct((NC * N_PAD, D_s), jnp.float32),
        mesh=_sc_mesh(),
        scratch_types=[pltpu.VMEM((CHUNKS, CH), jnp.int32),
                       pltpu.VMEM((CH, D_s), jnp.float32),
                       pltpu.SemaphoreType.DMA,
                       pltpu.VMEM_SHARED((N_PAD, D_s), jnp.float32)],
    )


def _sc_scatter(msgs, idx_flat, zer):
    return _sc_scatter_kernel()(msgs, idx_flat, zer)


def _tc5_body(s_ref, cid_ref, inc0_ref, inc1_ref,
              wu1a, wu1b, wu1c, bu1, wu2, bu2, out_ref):
    x = s_ref[...]
    cid = cid_ref[...]
    inc = inc0_ref[...] + inc1_ref[...]
    h = jax.nn.gelu(x @ wu1a[...] + cid @ wu1b[...] + inc @ wu1c[...] + bu1[...])
    out_ref[...] = x + h @ wu2[...] + bu2[...]


def _tc5(s_pad, cid_pad, inc0, inc1, wu1a, wu1b, wu1c, bu1, wu2, bu2):
    g = N_PAD // BN1
    row = lambda i: (i, 0)
    full = lambda i: (0, 0)
    wspec = lambda a: pl.BlockSpec(a.shape, full)
    return pl.pallas_call(
        _tc5_body,
        grid=(g,),
        in_specs=[pl.BlockSpec((BN1, D_s), row), pl.BlockSpec((BN1, D_id), row),
                  pl.BlockSpec((BN1, D_s), row), pl.BlockSpec((BN1, D_s), row)]
        + [wspec(a) for a in (wu1a, wu1b, wu1c, bu1, wu2, bu2)],
        out_specs=pl.BlockSpec((BN1, D_s), row),
        out_shape=jax.ShapeDtypeStruct((N_PAD, D_s), jnp.float32),
    )(s_pad, cid_pad, inc0, inc1, wu1a, wu1b, wu1c, bu1, wu2, bu2)


def kernel(s, out_nbrs, E_bias_flat, col_id, W_c1, b_c1, W_c2, b_c2,
           W_q1, b_q1, W_q2, b_q2, W_k1, b_k1, W_k2, b_k2,
           W_u1, b_u1, W_u2, b_u2):
    f32 = jnp.float32
    s2 = s[0].astype(f32)
    pad = N_PAD - N

    s_pad = jnp.pad(s2, ((0, pad), (0, 0)))
    cid_pad = jnp.pad(col_id.astype(f32), ((0, pad), (0, 0)))
    pad_idx = (jnp.arange(pad * K, dtype=jnp.int32) % N).reshape(pad, K)
    nbr_pad = jnp.concatenate([out_nbrs.astype(jnp.int32), pad_idx], axis=0)
    idx_flat = nbr_pad.T.reshape(IDXROWS, CH)
    eb_pad = jnp.pad(E_bias_flat.astype(f32).reshape(N, K), ((0, pad), (0, 0)))

    r2 = lambda b: b.reshape(1, -1).astype(f32)
    m, q, kvec = _tc1(
        s_pad, cid_pad,
        W_c1[:D_s].astype(f32), W_c1[D_s:].astype(f32), r2(b_c1),
        W_c2.astype(f32), r2(b_c2),
        W_q1[:D_s].astype(f32), W_q1[D_s:].astype(f32), r2(b_q1),
        W_q2.astype(f32), r2(b_q2),
        W_k1.astype(f32), r2(b_k1), W_k2.astype(f32), r2(b_k2))

    knbr_t = _sc_gather(kvec, idx_flat).reshape(K, N_PAD, HD)
    msgs = _tc3(q, knbr_t, eb_pad, m).reshape(E_PAD, D_s)

    zer = jnp.zeros((CH, D_s), f32)
    parts = _sc_scatter(msgs, idx_flat, zer)

    s_new = _tc5(s_pad, cid_pad, parts[:N_PAD], parts[N_PAD:],
                 W_u1[:D_s].astype(f32), W_u1[D_s:D_s + D_id].astype(f32),
                 W_u1[D_s + D_id:].astype(f32), r2(b_u1),
                 W_u2.astype(f32), r2(b_u2))
    return s_new[:N][None]


# speedup vs baseline: 1.1586x; 1.1586x over previous
import functools

import jax
import jax.numpy as jnp
from jax import lax
from jax.experimental import pallas as pl
from jax.experimental.pallas import tpu as pltpu
from jax.experimental.pallas import tpu_sc as plsc

N = 10000
K = 32
D_s = 128
D_id = 32
HD = 64
FF = 256

NC = 2
NS = 16
NW = NC * NS

N_PAD = 10240
E_PAD = N_PAD * K
EPW = E_PAD // NW
CH = 128
CHUNKS = EPW // CH
IDXROWS = E_PAD // CH
NPW = N_PAD // NW
REAL_ROWS = (N * K) // CH

MBLK = 32

BN1 = 512


@functools.cache
def _sc_mesh():
    return plsc.VectorSubcoreMesh(core_axis_name="c", subcore_axis_name="s",
                                  num_cores=NC, num_subcores=NS)


def _rms(x):
    return x * lax.rsqrt(jnp.mean(x * x, axis=-1, keepdims=True) + 1e-6)


def _sigmoid(x):
    return 1.0 / (1.0 + jnp.exp(-x))


def _tc1_body(s_ref, cid_ref, wc1a, wc1b, bc1, wc2, bc2,
              wq1a, wq1b, bq1, wq2, bq2, wk1, bk1, wk2, bk2,
              m_ref, q_ref, k_ref):
    x = s_ref[...]
    cid = cid_ref[...]
    sn = _rms(x)
    h = jax.nn.gelu(sn @ wc1a[...] + cid @ wc1b[...] + bc1[...])
    m_ref[...] = h @ wc2[...] + bc2[...]
    hq = jax.nn.gelu(x @ wq1a[...] + cid @ wq1b[...] + bq1[...])
    q_ref[...] = hq @ wq2[...] + bq2[...]
    hk = jax.nn.gelu(cid @ wk1[...] + bk1[...])
    k_ref[...] = hk @ wk2[...] + bk2[...]


def _tc1(s_pad, cid_pad, wc1a, wc1b, bc1, wc2, bc2,
         wq1a, wq1b, bq1, wq2, bq2, wk1, bk1, wk2, bk2):
    g = N_PAD // BN1
    row = lambda i: (i, 0)
    full = lambda i: (0, 0)
    wspec = lambda a: pl.BlockSpec(a.shape, full)
    return pl.pallas_call(
        _tc1_body,
        grid=(g,),
        in_specs=[pl.BlockSpec((BN1, D_s), row), pl.BlockSpec((BN1, D_id), row)]
        + [wspec(a) for a in (wc1a, wc1b, bc1, wc2, bc2,
                              wq1a, wq1b, bq1, wq2, bq2, wk1, bk1, wk2, bk2)],
        out_specs=[pl.BlockSpec((BN1, D_s), row),
                   pl.BlockSpec((BN1, HD), row),
                   pl.BlockSpec((BN1, HD), row)],
        out_shape=[jax.ShapeDtypeStruct((N_PAD, D_s), jnp.float32),
                   jax.ShapeDtypeStruct((N_PAD, HD), jnp.float32),
                   jax.ShapeDtypeStruct((N_PAD, HD), jnp.float32)],
    )(s_pad, cid_pad, wc1a, wc1b, bc1, wc2, bc2,
      wq1a, wq1b, bq1, wq2, bq2, wk1, bk1, wk2, bk2)


def _sc_score_body(k_hbm, q_hbm, idx_hbm, eb_hbm, w_out,
                   idx_v, eb_v, w_v, q_v, kr0, kr1, sg0, sg1):
    c = lax.axis_index("c")
    sid = lax.axis_index("s")
    wid = sid * NC + c
    base_row = wid * CHUNKS
    base_node = wid * NPW

    pltpu.sync_copy(idx_hbm.at[pl.ds(base_row, CHUNKS)], idx_v)
    pltpu.sync_copy(eb_hbm.at[pl.ds(base_row, CHUNKS)], eb_v)
    pltpu.sync_copy(q_hbm.at[pl.ds(base_node, NPW)], q_v)
    pltpu.async_copy(k_hbm.at[idx_v.at[0]], kr0, sg0)

    iota = lax.iota(jnp.int32, 16)

    def compute(j, kr):
        vf = jnp.where(base_row + j < REAL_ROWS, 1.0, 0.0)
        for n in range(4):
            acc0 = jnp.zeros((16,), jnp.float32)
            acc1 = jnp.zeros((16,), jnp.float32)
            r0 = iota + n * 32
            r1 = iota + n * 32 + 16
            qrow = jnp.full((16,), j * 4 + n, jnp.int32)
            for d in range(HD):
                dd = jnp.full((16,), d, jnp.int32)
                qd = plsc.load_gather(q_v, [qrow, dd])
                c0 = plsc.load_gather(kr, [r0, dd])
                c1 = plsc.load_gather(kr, [r1, dd])
                acc0 = acc0 + qd * c0
                acc1 = acc1 + qd * c1
            e0 = eb_v[j, pl.ds(n * 32, 16)]
            e1 = eb_v[j, pl.ds(n * 32 + 16, 16)]
            w_v[j, pl.ds(n * 32, 16)] = _sigmoid(acc0 + e0) * vf
            w_v[j, pl.ds(n * 32 + 16, 16)] = _sigmoid(acc1 + e1) * vf

    def step(j2, _):
        for p, (cur, nxt, scur, snxt) in enumerate(
                ((kr0, kr1, sg0, sg1), (kr1, kr0, sg1, sg0))):
            j = j2 * 2 + p
            pltpu.make_async_copy(k_hbm.at[idx_v.at[j]], cur, scur).wait()

            @pl.when(j + 1 < CHUNKS)
            def _():
                pltpu.async_copy(k_hbm.at[idx_v.at[j + 1]], nxt, snxt)

            compute(j, cur)
        return 0

    lax.fori_loop(0, CHUNKS // 2, step, 0)
    pltpu.sync_copy(w_v, w_out.at[pl.ds(base_row, CHUNKS)])


@functools.cache
def _sc_score_kernel():
    return pl.kernel(
        _sc_score_body,
        out_type=jax.ShapeDtypeStruct((IDXROWS, CH), jnp.float32),
        mesh=_sc_mesh(),
        scratch_types=[pltpu.VMEM((CHUNKS, CH), jnp.int32),
                       pltpu.VMEM((CHUNKS, CH), jnp.float32),
                       pltpu.VMEM((CHUNKS, CH), jnp.float32),
                       pltpu.VMEM((NPW, HD), jnp.float32),
                       pltpu.VMEM((CH, HD), jnp.float32),
                       pltpu.VMEM((CH, HD), jnp.float32),
                       pltpu.SemaphoreType.DMA,
                       pltpu.SemaphoreType.DMA],
        compiler_params=pltpu.CompilerParams(use_tc_tiling_on_sc=False,
                                             needs_layout_passes=False),
    )


def _sc_score(kvec, q, idx_flat, eb_flat):
    return _sc_score_kernel()(kvec, q, idx_flat, eb_flat)


def _sc_scatter2_body(m_hbm, w_hbm, dst_hbm, zer_hbm, out_hbm,
                      dst_v, wring, mblk, mg0, mg1, sg0, sg1, sw0, sw1, acc):
    c = lax.axis_index("c")
    sid = lax.axis_index("s")
    wid = sid * NC + c
    base_row = wid * CHUNKS
    base_node = wid * NPW
    rows_per_tile = N_PAD // NS
    zbase = sid * rows_per_tile

    pltpu.sync_copy(dst_hbm.at[pl.ds(base_row, CHUNKS)], dst_v)
    pltpu.async_copy(w_hbm.at[pl.ds(base_row, 1)], wring.at[pl.ds(0, 1)], sw0)
    for j in range(rows_per_tile // CH):
        pltpu.sync_copy(zer_hbm, acc.at[pl.ds(zbase + j * CH, CH)])
    plsc.subcore_barrier()

    def fill(j, buf, wslot):
        jm = lax.rem(j, 8)
        wj = jnp.full((16,), wslot, jnp.int32)
        for n in range(4):
            mrow = [mblk[jm * 4 + n, pl.ds(v * 16, 16)] for v in range(8)]

            def quad(q8, _):
                for t in range(4):
                    e = n * 32 + q8 * 4 + t
                    wb = plsc.load_gather(
                        wring, [wj, jnp.full((16,), e, jnp.int32)])
                    for v in range(8):
                        buf[e, pl.ds(v * 16, 16)] = wb * mrow[v]
                return 0

            lax.fori_loop(0, 8, quad, 0)

    def step(j2, _):
        for p, (buf, sg, sw, swn) in enumerate(
                ((mg0, sg0, sw0, sw1), (mg1, sg1, sw1, sw0))):
            j = j2 * 2 + p

            @pl.when(lax.rem(j, 8) == 0)
            def _():
                pltpu.sync_copy(
                    m_hbm.at[pl.ds(base_node + (j // 8) * MBLK, MBLK)], mblk)

            pltpu.make_async_copy(w_hbm.at[pl.ds(base_row, 1)],
                                  wring.at[pl.ds(p, 1)], sw).wait()

            @pl.when(j + 1 < CHUNKS)
            def _():
                pltpu.async_copy(w_hbm.at[pl.ds(base_row + j + 1, 1)],
                                 wring.at[pl.ds(1 - p, 1)], swn)

            @pl.when(j >= 2)
            def _():
                pltpu.make_async_copy(buf, acc.at[dst_v.at[j - 2]], sg).wait()

            fill(j, buf, p)
            pltpu.async_copy(buf, acc.at[dst_v.at[j]], sg, add=True)
        return 0

    lax.fori_loop(0, CHUNKS // 2, step, 0)
    pltpu.make_async_copy(mg0, acc.at[dst_v.at[CHUNKS - 2]], sg0).wait()
    pltpu.make_async_copy(mg1, acc.at[dst_v.at[CHUNKS - 1]], sg1).wait()
    plsc.subcore_barrier()

    for j in range(rows_per_tile // CH):
        r = zbase + j * CH
        pltpu.sync_copy(acc.at[pl.ds(r, CH)],
                        out_hbm.at[pl.ds(c * N_PAD + r, CH)])


@functools.cache
def _sc_scatter2_kernel():
    return pl.kernel(
        _sc_scatter2_body,
        out_type=jax.ShapeDtypeStruct((NC * N_PAD, D_s), jnp.float32),
        mesh=_sc_mesh(),
        scratch_types=[pltpu.VMEM((CHUNKS, CH), jnp.int32),
                       pltpu.VMEM((2, CH), jnp.float32),
                       pltpu.VMEM((MBLK, D_s), jnp.float32),
                       pltpu.VMEM((CH, D_s), jnp.float32),
                       pltpu.VMEM((CH, D_s), jnp.float32),
                       pltpu.SemaphoreType.DMA,
                       pltpu.SemaphoreType.DMA,
                       pltpu.SemaphoreType.DMA,
                       pltpu.SemaphoreType.DMA,
                       pltpu.VMEM_SHARED((N_PAD, D_s), jnp.float32)],
        compiler_params=pltpu.CompilerParams(needs_layout_passes=False),
    )


def _sc_scatter2(m, w2, idx2, zer):
    return _sc_scatter2_kernel()(m, w2, idx2, zer)


def _tc5_body(s_ref, cid_ref, inc0_ref, inc1_ref,
              wu1a, wu1b, wu1c, bu1, wu2, bu2, out_ref):
    x = s_ref[...]
    cid = cid_ref[...]
    inc = inc0_ref[...] + inc1_ref[...]
    h = jax.nn.gelu(x @ wu1a[...] + cid @ wu1b[...] + inc @ wu1c[...] + bu1[...])
    out_ref[...] = x + h @ wu2[...] + bu2[...]


def _tc5(s_pad, cid_pad, inc0, inc1, wu1a, wu1b, wu1c, bu1, wu2, bu2):
    g = N_PAD // BN1
    row = lambda i: (i, 0)
    full = lambda i: (0, 0)
    wspec = lambda a: pl.BlockSpec(a.shape, full)
    return pl.pallas_call(
        _tc5_body,
        grid=(g,),
        in_specs=[pl.BlockSpec((BN1, D_s), row), pl.BlockSpec((BN1, D_id), row),
                  pl.BlockSpec((BN1, D_s), row), pl.BlockSpec((BN1, D_s), row)]
        + [wspec(a) for a in (wu1a, wu1b, wu1c, bu1, wu2, bu2)],
        out_specs=pl.BlockSpec((BN1, D_s), row),
        out_shape=jax.ShapeDtypeStruct((N_PAD, D_s), jnp.float32),
    )(s_pad, cid_pad, inc0, inc1, wu1a, wu1b, wu1c, bu1, wu2, bu2)


def kernel(s, out_nbrs, E_bias_flat, col_id, W_c1, b_c1, W_c2, b_c2,
           W_q1, b_q1, W_q2, b_q2, W_k1, b_k1, W_k2, b_k2,
           W_u1, b_u1, W_u2, b_u2):
    f32 = jnp.float32
    s2 = s[0].astype(f32)
    pad = N_PAD - N

    s_pad = jnp.pad(s2, ((0, pad), (0, 0)))
    cid_pad = jnp.pad(col_id.astype(f32), ((0, pad), (0, 0)))
    pad_idx = (jnp.arange(pad * K, dtype=jnp.int32) % N).reshape(pad, K)
    nbr_pad = jnp.concatenate([out_nbrs.astype(jnp.int32), pad_idx], axis=0)
    idx_flat = nbr_pad.reshape(IDXROWS, CH)
    eb_pad = jnp.pad(E_bias_flat.astype(f32).reshape(N, K), ((0, pad), (0, 0)))
    eb_flat = eb_pad.reshape(IDXROWS, CH)

    r2 = lambda b: b.reshape(1, -1).astype(f32)
    m, q, kvec = _tc1(
        s_pad, cid_pad,
        W_c1[:D_s].astype(f32), W_c1[D_s:].astype(f32), r2(b_c1),
        W_c2.astype(f32), r2(b_c2),
        W_q1[:D_s].astype(f32), W_q1[D_s:].astype(f32), r2(b_q1),
        W_q2.astype(f32), r2(b_q2),
        W_k1.astype(f32), r2(b_k1), W_k2.astype(f32), r2(b_k2))

    w = _sc_score(kvec, q, idx_flat, eb_flat)

    zer = jnp.zeros((CH, D_s), f32)
    parts = _sc_scatter2(m, w, idx_flat, zer)

    s_new = _tc5(s_pad, cid_pad, parts[:N_PAD], parts[N_PAD:],
                 W_u1[:D_s].astype(f32), W_u1[D_s:D_s + D_id].astype(f32),
                 W_u1[D_s + D_id:].astype(f32), r2(b_u1),
                 W_u2.astype(f32), r2(b_u2))
    return s_new[:N][None]
